# baseline (speedup 1.0000x reference)
import functools

import jax
import jax.numpy as jnp
from jax import lax
from jax.experimental import pallas as pl
from jax.experimental.pallas import tpu as pltpu
from jax.experimental.pallas import tpu_sc as plsc

B = 16384
EMB = 32
HID = 64
NV = 1000000
NC = 2
NS = 16
NW = NC * NS
BPW = B // NW
L = 16

BLKN = 40960


def _score_body(ut_ref, it_ref, w1_ref, w2_ref, b1_ref, b2_ref, su_ref, si_ref):
    fold = jnp.dot(w2_ref[...], w1_ref[...],
                   preferred_element_type=jnp.float32)
    v1 = fold[:, :EMB]
    v2 = fold[:, EMB:]
    c = jnp.sum(b1_ref[...] * w2_ref[...]) + b2_ref[0, 0]
    su = jnp.dot(v1, ut_ref[...], preferred_element_type=jnp.float32) + c
    si = jnp.dot(v2, it_ref[...], preferred_element_type=jnp.float32)
    su_ref[...] = su.reshape(BLKN)
    si_ref[...] = si.reshape(BLKN)


_score = pl.pallas_call(
    _score_body,
    grid=(pl.cdiv(NV, BLKN),),
    in_specs=[
        pl.BlockSpec((EMB, BLKN), lambda i: (0, i)),
        pl.BlockSpec((EMB, BLKN), lambda i: (0, i)),
        pl.BlockSpec((HID, HID), lambda i: (0, 0)),
        pl.BlockSpec((1, HID), lambda i: (0, 0)),
        pl.BlockSpec((1, HID), lambda i: (0, 0)),
        pl.BlockSpec((1, 1), lambda i: (0, 0)),
    ],
    out_specs=[
        pl.BlockSpec((BLKN,), lambda i: (i,)),
        pl.BlockSpec((BLKN,), lambda i: (i,)),
    ],
    out_shape=[
        jax.ShapeDtypeStruct((NV,), jnp.float32),
        jax.ShapeDtypeStruct((NV,), jnp.float32),
    ],
)


_mesh = plsc.VectorSubcoreMesh(core_axis_name="c", subcore_axis_name="s")


@functools.partial(
    pl.kernel,
    mesh=_mesh,
    compiler_params=pltpu.CompilerParams(use_tc_tiling_on_sc=False),
    out_type=jax.ShapeDtypeStruct((B,), jnp.float32),
    scratch_types=[
        pltpu.VMEM((BPW,), jnp.int32),
        pltpu.VMEM((BPW,), jnp.int32),
        pltpu.VMEM((BPW,), jnp.float32),
        pltpu.VMEM((BPW,), jnp.float32),
        pltpu.VMEM((BPW,), jnp.float32),
        pltpu.SemaphoreType.DMA,
        pltpu.SemaphoreType.DMA,
    ],
)
def _sc_gather_sigmoid(users_hbm, items_hbm, su_hbm, si_hbm, out_hbm,
                       idxu_v, idxi_v, su_v, si_v, out_v, semu, semi):
    wid = lax.axis_index("s") * NC + lax.axis_index("c")
    base = wid * BPW
    pltpu.sync_copy(users_hbm.at[pl.ds(base, BPW)], idxu_v)
    pltpu.sync_copy(items_hbm.at[pl.ds(base, BPW)], idxi_v)
    cu = pltpu.async_copy(su_hbm.at[idxu_v], su_v, semu)
    ci = pltpu.async_copy(si_hbm.at[idxi_v], si_v, semi)
    cu.wait()
    ci.wait()
    def sig_body(g, carry):
        s = su_v[pl.ds(g * L, L)] + si_v[pl.ds(g * L, L)]
        out_v[pl.ds(g * L, L)] = 1.0 / (1.0 + jnp.exp(-s))
        return carry

    lax.fori_loop(0, BPW // L, sig_body, 0)
    pltpu.sync_copy(out_v, out_hbm.at[pl.ds(base, BPW)])


def kernel(users, items, user_emb, item_emb, W1, b1, W2, b2):
    users = users.astype(jnp.int32)
    items = items.astype(jnp.int32)
    su, si = _score(user_emb.T, item_emb.T, W1, W2, b1.reshape(1, HID),
                    b2.reshape(1, 1))
    out = _sc_gather_sigmoid(users, items, su, si)
    return out.reshape(B, 1)


# speedup vs baseline: 1.0071x; 1.0071x over previous
import functools

import jax
import jax.numpy as jnp
from jax import lax
from jax.experimental import pallas as pl
from jax.experimental.pallas import tpu as pltpu
from jax.experimental.pallas import tpu_sc as plsc

B = 16384
EMB = 32
HID = 64
NV = 1000000
NC = 2
NS = 16
NW = NC * NS
BPW = B // NW
L = 16

BLKN = 32768


def _score_body(ut_ref, it_ref, w1_ref, w2_ref, b1_ref, b2_ref, su_ref, si_ref):
    fold = jnp.dot(w2_ref[...], w1_ref[...],
                   preferred_element_type=jnp.float32)
    v1 = fold[:, :EMB]
    v2 = fold[:, EMB:]
    c = jnp.sum(b1_ref[...] * w2_ref[...]) + b2_ref[0, 0]
    su = jnp.dot(v1, ut_ref[...], preferred_element_type=jnp.float32) + c
    si = jnp.dot(v2, it_ref[...], preferred_element_type=jnp.float32)
    su_ref[...] = su.reshape(BLKN)
    si_ref[...] = si.reshape(BLKN)


_score = pl.pallas_call(
    _score_body,
    grid=(pl.cdiv(NV, BLKN),),
    in_specs=[
        pl.BlockSpec((EMB, BLKN), lambda i: (0, i)),
        pl.BlockSpec((EMB, BLKN), lambda i: (0, i)),
        pl.BlockSpec((HID, HID), lambda i: (0, 0)),
        pl.BlockSpec((1, HID), lambda i: (0, 0)),
        pl.BlockSpec((1, HID), lambda i: (0, 0)),
        pl.BlockSpec((1, 1), lambda i: (0, 0)),
    ],
    out_specs=[
        pl.BlockSpec((BLKN,), lambda i: (i,)),
        pl.BlockSpec((BLKN,), lambda i: (i,)),
    ],
    out_shape=[
        jax.ShapeDtypeStruct((NV,), jnp.float32),
        jax.ShapeDtypeStruct((NV,), jnp.float32),
    ],
)


_mesh = plsc.VectorSubcoreMesh(core_axis_name="c", subcore_axis_name="s")


@functools.partial(
    pl.kernel,
    mesh=_mesh,
    compiler_params=pltpu.CompilerParams(use_tc_tiling_on_sc=False),
    out_type=jax.ShapeDtypeStruct((B,), jnp.float32),
    scratch_types=[
        pltpu.VMEM((BPW,), jnp.int32),
        pltpu.VMEM((BPW,), jnp.int32),
        pltpu.VMEM((BPW,), jnp.float32),
        pltpu.VMEM((BPW,), jnp.float32),
        pltpu.VMEM((BPW,), jnp.float32),
        pltpu.SemaphoreType.DMA,
        pltpu.SemaphoreType.DMA,
    ],
)
def _sc_gather_sigmoid(users_hbm, items_hbm, su_hbm, si_hbm, out_hbm,
                       idxu_v, idxi_v, su_v, si_v, out_v, semu, semi):
    wid = lax.axis_index("s") * NC + lax.axis_index("c")
    base = wid * BPW
    pltpu.sync_copy(users_hbm.at[pl.ds(base, BPW)], idxu_v)
    pltpu.sync_copy(items_hbm.at[pl.ds(base, BPW)], idxi_v)
    cu = pltpu.async_copy(su_hbm.at[idxu_v], su_v, semu)
    ci = pltpu.async_copy(si_hbm.at[idxi_v], si_v, semi)
    cu.wait()
    ci.wait()
    def sig_body(g, carry):
        s = su_v[pl.ds(g * L, L)] + si_v[pl.ds(g * L, L)]
        out_v[pl.ds(g * L, L)] = 1.0 / (1.0 + jnp.exp(-s))
        return carry

    lax.fori_loop(0, BPW // L, sig_body, 0)
    pltpu.sync_copy(out_v, out_hbm.at[pl.ds(base, BPW)])


def kernel(users, items, user_emb, item_emb, W1, b1, W2, b2):
    users = users.astype(jnp.int32)
    items = items.astype(jnp.int32)
    su, si = _score(user_emb.T, item_emb.T, W1, W2, b1.reshape(1, HID),
                    b2.reshape(1, 1))
    out = _sc_gather_sigmoid(users, items, su, si)
    return out.reshape(B, 1)
